# baseline (speedup 1.0000x reference)
import functools

import jax
import jax.numpy as jnp
from jax import lax
from jax.experimental import pallas as pl
from jax.experimental.pallas import tpu as pltpu
from jax.experimental.pallas import tpu_sc as plsc

B = 16384
D = 16
H = 50
NC = 2
NS = 16
NW = NC * NS
BPW = B // NW
HP = 128
HG = 56
C = 32
NCH = BPW // C
IPC = C * HG
GSZ = 128

_mesh = plsc.VectorSubcoreMesh(
    core_axis_name="c", subcore_axis_name="s", num_cores=NC, num_subcores=NS
)


@functools.partial(
    pl.kernel,
    out_type=(
        jax.ShapeDtypeStruct((B, D), jnp.float32),
        jax.ShapeDtypeStruct((B, D), jnp.float32),
    ),
    mesh=_mesh,
    compiler_params=pltpu.CompilerParams(use_tc_tiling_on_sc=False),
    scratch_types=[
        pltpu.VMEM((BPW,), jnp.int32),
        pltpu.VMEM((BPW, D), jnp.float32),
        pltpu.VMEM((C, HP), jnp.int32),
        pltpu.VMEM((C, HP), jnp.int32),
        pltpu.VMEM((IPC, D), jnp.float32),
        pltpu.VMEM((IPC, D), jnp.float32),
        pltpu.VMEM((C, D), jnp.float32),
        pltpu.SemaphoreType.DMA,
        pltpu.SemaphoreType.DMA,
        pltpu.SemaphoreType.DMA,
    ],
)
def _sc_gather_mean(ds_idx_hbm, lab_idx_hbm, ds_tab, lab_tab, ds_out, lab_out,
                    ds_idx_v, ds_rows_v, li0, li1, rb0, rb1, acc_v,
                    sem_ds, sem0, sem1):
    wid = lax.axis_index("s") * NC + lax.axis_index("c")
    base = wid * BPW

    pltpu.sync_copy(ds_idx_hbm.at[pl.ds(base, BPW)], ds_idx_v)
    for g in range(BPW // GSZ):
        pltpu.async_copy(
            ds_tab.at[ds_idx_v.at[pl.ds(g * GSZ, GSZ)]],
            ds_rows_v.at[pl.ds(g * GSZ, GSZ)],
            sem_ds,
        )

    idx_bufs = (li0, li1)
    row_bufs = (rb0, rb1)
    sems = (sem0, sem1)

    def fire(c, buf):
        pltpu.sync_copy(
            lab_idx_hbm.at[pl.ds(base + c * C, C), :], idx_bufs[buf]
        )

        def gbody(b, carry):
            pltpu.async_copy(
                lab_tab.at[idx_bufs[buf].at[b, pl.ds(0, HG)]],
                row_bufs[buf].at[pl.ds(b * HG, HG)],
                sems[buf],
            )
            return carry

        lax.fori_loop(0, C, gbody, 0)

    def drain(buf):
        pltpu.make_async_copy(
            lab_tab.at[pl.ds(0, IPC)], row_bufs[buf], sems[buf]
        ).wait()

    def reduce(c, buf):
        rows = row_bufs[buf]

        def rbody(b, carry):
            r0 = b * HG
            s = [rows[r0 + k, :] for k in range(4)]
            for j in range(4, H):
                s[j % 4] = s[j % 4] + rows[r0 + j, :]
            acc_v[b, :] = ((s[0] + s[1]) + (s[2] + s[3])) * (1.0 / H)
            return carry

        lax.fori_loop(0, C, rbody, 0)
        pltpu.sync_copy(acc_v, lab_out.at[pl.ds(base + c * C, C)])

    fire(0, 0)
    for c in range(NCH):
        if c + 1 < NCH:
            fire(c + 1, (c + 1) % 2)
        drain(c % 2)
        reduce(c, c % 2)

    pltpu.make_async_copy(ds_tab.at[pl.ds(0, BPW)], ds_rows_v, sem_ds).wait()
    pltpu.sync_copy(ds_rows_v, ds_out.at[pl.ds(base, BPW)])


BLK = 1024


def _pad_body(x_ref, o_ref):
    x = x_ref[...]
    z = jnp.zeros((x.shape[0], HP - H), jnp.int32)
    o_ref[...] = jnp.concatenate([x, z], axis=1)


def _mlp_body(ds_ref, lab_ref, num_ref, w1_ref, b1_ref, w2_ref, b2_ref, o_ref):
    nf = lax.dot_general(
        num_ref[...], w1_ref[...], (((1,), (1,)), ((), ())),
        preferred_element_type=jnp.float32,
    ) + b1_ref[...]
    w2 = w2_ref[...]
    out = lax.dot_general(
        ds_ref[...], w2[:, 0:D], (((1,), (1,)), ((), ())),
        preferred_element_type=jnp.float32,
    )
    out = out + lax.dot_general(
        lab_ref[...], w2[:, D:2 * D], (((1,), (1,)), ((), ())),
        preferred_element_type=jnp.float32,
    )
    out = out + lax.dot_general(
        nf, w2[:, 2 * D:], (((1,), (1,)), ((), ())),
        preferred_element_type=jnp.float32,
    )
    o_ref[...] = jnp.maximum(out + b2_ref[...], 0.0)


def kernel(dataset_name, unique_labels, numeric_features, dataset_name_table,
           labels_table, W1, b1, W2, b2):
    ds_idx = dataset_name.astype(jnp.int32)
    lab_idx = pl.pallas_call(
        _pad_body,
        grid=(B // BLK,),
        in_specs=[pl.BlockSpec((BLK, H), lambda i: (i, 0))],
        out_specs=pl.BlockSpec((BLK, HP), lambda i: (i, 0)),
        out_shape=jax.ShapeDtypeStruct((B, HP), jnp.int32),
    )(unique_labels.astype(jnp.int32))

    ds_emb, lab_mean = _sc_gather_mean(
        ds_idx, lab_idx, dataset_name_table, labels_table
    )

    n_in = W1.shape[1]
    h1 = W1.shape[0]
    n_out = W2.shape[0]
    out = pl.pallas_call(
        _mlp_body,
        grid=(B // BLK,),
        in_specs=[
            pl.BlockSpec((BLK, D), lambda i: (i, 0)),
            pl.BlockSpec((BLK, D), lambda i: (i, 0)),
            pl.BlockSpec((BLK, n_in), lambda i: (i, 0)),
            pl.BlockSpec((h1, n_in), lambda i: (0, 0)),
            pl.BlockSpec((1, h1), lambda i: (0, 0)),
            pl.BlockSpec((n_out, h1 + 2 * D), lambda i: (0, 0)),
            pl.BlockSpec((1, n_out), lambda i: (0, 0)),
        ],
        out_specs=pl.BlockSpec((BLK, n_out), lambda i: (i, 0)),
        out_shape=jax.ShapeDtypeStruct((B, n_out), jnp.float32),
    )(ds_emb, lab_mean, numeric_features, W1, b1.reshape(1, h1), W2,
      b2.reshape(1, n_out))
    return out


# speedup vs baseline: 1.8590x; 1.8590x over previous
import functools

import jax
import jax.numpy as jnp
from jax import lax
from jax.experimental import pallas as pl
from jax.experimental.pallas import tpu as pltpu
from jax.experimental.pallas import tpu_sc as plsc

B = 16384
D = 16
H = 50
NC = 2
NS = 16
NW = NC * NS
BPW = B // NW
C = 64
NCH = BPW // C
IPC = C * H
GSZ = 128
NG = IPC // GSZ

_mesh = plsc.VectorSubcoreMesh(
    core_axis_name="c", subcore_axis_name="s", num_cores=NC, num_subcores=NS
)


@functools.partial(
    pl.kernel,
    out_type=(
        jax.ShapeDtypeStruct((B, D), jnp.float32),
        jax.ShapeDtypeStruct((B, D), jnp.float32),
    ),
    mesh=_mesh,
    compiler_params=pltpu.CompilerParams(use_tc_tiling_on_sc=False),
    scratch_types=[
        pltpu.VMEM((BPW,), jnp.int32),
        pltpu.VMEM((BPW, D), jnp.float32),
        pltpu.VMEM((IPC,), jnp.int32),
        pltpu.VMEM((IPC,), jnp.int32),
        pltpu.VMEM((IPC, D), jnp.float32),
        pltpu.VMEM((IPC, D), jnp.float32),
        pltpu.VMEM((C, D), jnp.float32),
        pltpu.SemaphoreType.DMA,
        pltpu.SemaphoreType.DMA,
        pltpu.SemaphoreType.DMA,
    ],
)
def _sc_gather_mean(ds_idx_hbm, lab_idx_hbm, ds_tab, lab_tab, ds_out, lab_out,
                    ds_idx_v, ds_rows_v, li0, li1, rb0, rb1, acc_v,
                    sem_ds, sem0, sem1):
    wid = lax.axis_index("s") * NC + lax.axis_index("c")
    base = wid * BPW
    lab_base = base * H

    pltpu.sync_copy(ds_idx_hbm.at[pl.ds(base, BPW)], ds_idx_v)
    for g in range(BPW // GSZ):
        pltpu.async_copy(
            ds_tab.at[ds_idx_v.at[pl.ds(g * GSZ, GSZ)]],
            ds_rows_v.at[pl.ds(g * GSZ, GSZ)],
            sem_ds,
        )

    idx_bufs = (li0, li1)
    row_bufs = (rb0, rb1)
    sems = (sem0, sem1)

    def fire(c, buf):
        pltpu.sync_copy(
            lab_idx_hbm.at[pl.ds(lab_base + c * IPC, IPC)], idx_bufs[buf]
        )

        def body(g, carry):
            pltpu.async_copy(
                lab_tab.at[idx_bufs[buf].at[pl.ds(g * GSZ, GSZ)]],
                row_bufs[buf].at[pl.ds(g * GSZ, GSZ)],
                sems[buf],
            )
            return carry

        lax.fori_loop(0, NG, body, 0)

    def drain(buf):
        pltpu.make_async_copy(
            lab_tab.at[pl.ds(0, IPC)], row_bufs[buf], sems[buf]
        ).wait()

    def reduce(c, buf):
        rows = row_bufs[buf]

        def rbody(b, carry):
            r0 = b * H
            s = [rows[r0 + k, :] for k in range(4)]
            for j in range(4, H):
                s[j % 4] = s[j % 4] + rows[r0 + j, :]
            acc_v[b, :] = ((s[0] + s[1]) + (s[2] + s[3])) * (1.0 / H)
            return carry

        lax.fori_loop(0, C, rbody, 0)
        pltpu.sync_copy(acc_v, lab_out.at[pl.ds(base + c * C, C)])

    fire(0, 0)
    for c in range(NCH):
        if c + 1 < NCH:
            fire(c + 1, (c + 1) % 2)
        drain(c % 2)
        reduce(c, c % 2)

    pltpu.make_async_copy(ds_tab.at[pl.ds(0, BPW)], ds_rows_v, sem_ds).wait()
    pltpu.sync_copy(ds_rows_v, ds_out.at[pl.ds(base, BPW)])


BLK = 1024


def _mlp_body(ds_ref, lab_ref, num_ref, w1_ref, b1_ref, w2_ref, b2_ref, o_ref):
    nf = lax.dot_general(
        num_ref[...], w1_ref[...], (((1,), (1,)), ((), ())),
        preferred_element_type=jnp.float32,
    ) + b1_ref[...]
    w2 = w2_ref[...]
    out = lax.dot_general(
        ds_ref[...], w2[:, 0:D], (((1,), (1,)), ((), ())),
        preferred_element_type=jnp.float32,
    )
    out = out + lax.dot_general(
        lab_ref[...], w2[:, D:2 * D], (((1,), (1,)), ((), ())),
        preferred_element_type=jnp.float32,
    )
    out = out + lax.dot_general(
        nf, w2[:, 2 * D:], (((1,), (1,)), ((), ())),
        preferred_element_type=jnp.float32,
    )
    o_ref[...] = jnp.maximum(out + b2_ref[...], 0.0)


def kernel(dataset_name, unique_labels, numeric_features, dataset_name_table,
           labels_table, W1, b1, W2, b2):
    ds_idx = dataset_name.astype(jnp.int32)
    lab_idx = unique_labels.astype(jnp.int32).reshape(-1)

    ds_emb, lab_mean = _sc_gather_mean(
        ds_idx, lab_idx, dataset_name_table, labels_table
    )

    n_in = W1.shape[1]
    h1 = W1.shape[0]
    n_out = W2.shape[0]
    out = pl.pallas_call(
        _mlp_body,
        grid=(B // BLK,),
        in_specs=[
            pl.BlockSpec((BLK, D), lambda i: (i, 0)),
            pl.BlockSpec((BLK, D), lambda i: (i, 0)),
            pl.BlockSpec((BLK, n_in), lambda i: (i, 0)),
            pl.BlockSpec((h1, n_in), lambda i: (0, 0)),
            pl.BlockSpec((1, h1), lambda i: (0, 0)),
            pl.BlockSpec((n_out, h1 + 2 * D), lambda i: (0, 0)),
            pl.BlockSpec((1, n_out), lambda i: (0, 0)),
        ],
        out_specs=pl.BlockSpec((BLK, n_out), lambda i: (i, 0)),
        out_shape=jax.ShapeDtypeStruct((B, n_out), jnp.float32),
    )(ds_emb, lab_mean, numeric_features, W1, b1.reshape(1, h1), W2,
      b2.reshape(1, n_out))
    return out
